# baseline (speedup 1.0000x reference)
import functools

import jax
import jax.numpy as jnp
from jax import lax
from jax.experimental import pallas as pl
from jax.experimental.pallas import tpu as pltpu
from jax.experimental.pallas import tpu_sc as plsc

_TOP_K = 8
_NUM_EXPERTS = 64
_HIDDEN = 4096
_N_TOKENS = 8192
_BLOCK_M = 1024

_NC = 2
_NS = 16
_L = 16
_NW = _NC * _NS
_ROWS_PER_W = _N_TOKENS // _NW
_GROUPS = _ROWS_PER_W // _L


def _router_body(x_ref, wt_ref, probs_ref, probs_t_ref):
    x = x_ref[...]
    wt = wt_ref[...]
    logits = jax.lax.dot_general(
        x, wt, (((1,), (0,)), ((), ())), preferred_element_type=jnp.float32
    )
    m = jnp.max(logits, axis=-1, keepdims=True)
    e = jnp.exp(logits - m)
    probs = e / jnp.sum(e, axis=-1, keepdims=True)
    probs_ref[...] = probs
    probs_t_ref[...] = probs.T


def _tc_softmax_router(x, wt):
    grid = (_N_TOKENS // _BLOCK_M,)
    return pl.pallas_call(
        _router_body,
        grid=grid,
        in_specs=[
            pl.BlockSpec((_BLOCK_M, _HIDDEN), lambda i: (i, 0)),
            pl.BlockSpec((_HIDDEN, _NUM_EXPERTS), lambda i: (0, 0)),
        ],
        out_specs=[
            pl.BlockSpec((_BLOCK_M, _NUM_EXPERTS), lambda i: (i, 0)),
            pl.BlockSpec((_NUM_EXPERTS, _BLOCK_M), lambda i: (0, i)),
        ],
        out_shape=[
            jax.ShapeDtypeStruct((_N_TOKENS, _NUM_EXPERTS), jnp.float32),
            jax.ShapeDtypeStruct((_NUM_EXPERTS, _N_TOKENS), jnp.float32),
        ],
    )(x, wt)


_sc_mesh = plsc.VectorSubcoreMesh(
    core_axis_name="c", subcore_axis_name="s", num_cores=_NC, num_subcores=_NS
)


@functools.partial(
    pl.kernel,
    out_type=[
        jax.ShapeDtypeStruct((_TOP_K, _N_TOKENS), jnp.float32),
        jax.ShapeDtypeStruct((_TOP_K, _N_TOKENS), jnp.int32),
    ],
    mesh=_sc_mesh,
    scratch_types=[
        pltpu.VMEM((_NUM_EXPERTS, _ROWS_PER_W), jnp.float32),
        pltpu.VMEM((_TOP_K, _ROWS_PER_W), jnp.float32),
        pltpu.VMEM((_TOP_K, _ROWS_PER_W), jnp.int32),
        pltpu.SemaphoreType.DMA,
        pltpu.SemaphoreType.DMA,
    ],
)
def _sc_topk(probs_t_hbm, out_v_hbm, out_i_hbm, slab, val_buf, idx_buf, s0, s1):
    wid = lax.axis_index("s") * _NC + lax.axis_index("c")
    base = wid * _ROWS_PER_W
    half = _ROWS_PER_W // 2
    c0 = pltpu.async_copy(
        probs_t_hbm.at[:, pl.ds(base, half)], slab.at[:, pl.ds(0, half)], s0
    )
    c1 = pltpu.async_copy(
        probs_t_hbm.at[:, pl.ds(base + half, half)],
        slab.at[:, pl.ds(half, half)],
        s1,
    )
    c0.wait()

    def group_body(g, carry):
        col = g * _L

        mxs = []
        total = jnp.zeros((_L,), jnp.float32)
        mx = None
        for r in range(_TOP_K):
            vs = [slab[e, pl.ds(col, _L)] for e in range(_NUM_EXPERTS)]
            if r > 0:
                vs = [jnp.where(v < mx, v, -1.0) for v in vs]
            ids = list(range(_NUM_EXPERTS))
            lvl_v = [
                jnp.maximum(vs[i], vs[i + 1]) for i in range(0, _NUM_EXPERTS, 2)
            ]
            lvl_i = [
                jnp.where(vs[i] >= vs[i + 1], ids[i], ids[i + 1])
                for i in range(0, _NUM_EXPERTS, 2)
            ]
            while len(lvl_v) > 1:
                nv, ni = [], []
                for i in range(0, len(lvl_v), 2):
                    ge = lvl_v[i] >= lvl_v[i + 1]
                    nv.append(jnp.maximum(lvl_v[i], lvl_v[i + 1]))
                    ni.append(jnp.where(ge, lvl_i[i], lvl_i[i + 1]))
                lvl_v, lvl_i = nv, ni
            mx, ix = lvl_v[0], lvl_i[0]
            idx_buf[r, pl.ds(col, _L)] = ix
            mxs.append(mx)
            total = total + mx

        inv = 1.0 / total
        for r in range(_TOP_K):
            val_buf[r, pl.ds(col, _L)] = mxs[r] * inv
        return carry

    lax.fori_loop(0, _GROUPS // 2, group_body, 0)
    c1.wait()
    lax.fori_loop(_GROUPS // 2, _GROUPS, group_body, 0)
    pltpu.sync_copy(val_buf, out_v_hbm.at[:, pl.ds(base, _ROWS_PER_W)])
    pltpu.sync_copy(idx_buf, out_i_hbm.at[:, pl.ds(base, _ROWS_PER_W)])


@jax.jit
def kernel(hidden_states, weight):
    x = hidden_states.reshape(-1, _HIDDEN)
    wt = weight.T
    probs, probs_t = _tc_softmax_router(x, wt)
    vals_t, idx_t = _sc_topk(probs_t)
    return (probs, vals_t.T, idx_t.T)


# speedup vs baseline: 1.0216x; 1.0216x over previous
import functools

import jax
import jax.numpy as jnp
from jax import lax
from jax.experimental import pallas as pl
from jax.experimental.pallas import tpu as pltpu
from jax.experimental.pallas import tpu_sc as plsc

_TOP_K = 8
_NUM_EXPERTS = 64
_HIDDEN = 4096
_N_TOKENS = 8192
_BLOCK_M = 1024

_NC = 2
_NS = 16
_L = 16
_NW = _NC * _NS
_ROWS_PER_W = _N_TOKENS // _NW
_GROUPS = _ROWS_PER_W // _L


def _router_body(x_ref, wt_ref, probs_ref, probs_t_ref):
    x = x_ref[...]
    wt = wt_ref[...]
    logits = jax.lax.dot_general(
        x, wt, (((1,), (0,)), ((), ())), preferred_element_type=jnp.float32
    )
    m = jnp.max(logits, axis=-1, keepdims=True)
    e = jnp.exp(logits - m)
    probs = e / jnp.sum(e, axis=-1, keepdims=True)
    probs_ref[...] = probs
    nw_blk = _BLOCK_M // _ROWS_PER_W
    probs_t_ref[...] = probs.reshape(
        nw_blk, _ROWS_PER_W, _NUM_EXPERTS
    ).transpose(0, 2, 1)


def _tc_softmax_router(x, wt):
    grid = (_N_TOKENS // _BLOCK_M,)
    return pl.pallas_call(
        _router_body,
        grid=grid,
        in_specs=[
            pl.BlockSpec((_BLOCK_M, _HIDDEN), lambda i: (i, 0)),
            pl.BlockSpec((_HIDDEN, _NUM_EXPERTS), lambda i: (0, 0)),
        ],
        out_specs=[
            pl.BlockSpec((_BLOCK_M, _NUM_EXPERTS), lambda i: (i, 0)),
            pl.BlockSpec(
                (_BLOCK_M // _ROWS_PER_W, _NUM_EXPERTS, _ROWS_PER_W),
                lambda i: (i, 0, 0),
            ),
        ],
        out_shape=[
            jax.ShapeDtypeStruct((_N_TOKENS, _NUM_EXPERTS), jnp.float32),
            jax.ShapeDtypeStruct(
                (_NW, _NUM_EXPERTS, _ROWS_PER_W), jnp.float32
            ),
        ],
    )(x, wt)


_sc_mesh = plsc.VectorSubcoreMesh(
    core_axis_name="c", subcore_axis_name="s", num_cores=_NC, num_subcores=_NS
)


@functools.partial(
    pl.kernel,
    out_type=[
        jax.ShapeDtypeStruct((_TOP_K, _N_TOKENS), jnp.float32),
        jax.ShapeDtypeStruct((_TOP_K, _N_TOKENS), jnp.int32),
    ],
    mesh=_sc_mesh,
    scratch_types=[
        pltpu.VMEM((_NUM_EXPERTS, _ROWS_PER_W), jnp.float32),
        pltpu.VMEM((_TOP_K, _ROWS_PER_W), jnp.float32),
        pltpu.VMEM((_TOP_K, _ROWS_PER_W), jnp.int32),
    ],
)
def _sc_topk(probs_t_hbm, out_v_hbm, out_i_hbm, slab, val_buf, idx_buf):
    wid = lax.axis_index("s") * _NC + lax.axis_index("c")
    base = wid * _ROWS_PER_W
    pltpu.sync_copy(probs_t_hbm.at[wid], slab)

    def group_body(g, carry):
        col = g * _L

        mxs = []
        total = jnp.zeros((_L,), jnp.float32)
        mx = None
        for r in range(_TOP_K):
            vs = [slab[e, pl.ds(col, _L)] for e in range(_NUM_EXPERTS)]
            if r > 0:
                vs = [jnp.where(v < mx, v, -1.0) for v in vs]
            ids = list(range(_NUM_EXPERTS))
            lvl_v = [
                jnp.maximum(vs[i], vs[i + 1]) for i in range(0, _NUM_EXPERTS, 2)
            ]
            lvl_i = [
                jnp.where(vs[i] >= vs[i + 1], ids[i], ids[i + 1])
                for i in range(0, _NUM_EXPERTS, 2)
            ]
            while len(lvl_v) > 1:
                nv, ni = [], []
                for i in range(0, len(lvl_v), 2):
                    ge = lvl_v[i] >= lvl_v[i + 1]
                    nv.append(jnp.maximum(lvl_v[i], lvl_v[i + 1]))
                    ni.append(jnp.where(ge, lvl_i[i], lvl_i[i + 1]))
                lvl_v, lvl_i = nv, ni
            mx, ix = lvl_v[0], lvl_i[0]
            idx_buf[r, pl.ds(col, _L)] = ix
            mxs.append(mx)
            total = total + mx

        inv = 1.0 / total
        for r in range(_TOP_K):
            val_buf[r, pl.ds(col, _L)] = mxs[r] * inv
        return carry

    lax.fori_loop(0, _GROUPS, group_body, 0)
    pltpu.sync_copy(val_buf, out_v_hbm.at[:, pl.ds(base, _ROWS_PER_W)])
    pltpu.sync_copy(idx_buf, out_i_hbm.at[:, pl.ds(base, _ROWS_PER_W)])


@jax.jit
def kernel(hidden_states, weight):
    x = hidden_states.reshape(-1, _HIDDEN)
    wt = weight.T
    probs, probs_t = _tc_softmax_router(x, wt)
    vals_t, idx_t = _sc_topk(probs_t)
    return (probs, vals_t.T, idx_t.T)


# speedup vs baseline: 1.0229x; 1.0013x over previous
import functools

import jax
import jax.numpy as jnp
from jax import lax
from jax.experimental import pallas as pl
from jax.experimental.pallas import tpu as pltpu
from jax.experimental.pallas import tpu_sc as plsc

_TOP_K = 8
_NUM_EXPERTS = 64
_HIDDEN = 4096
_N_TOKENS = 8192
_BLOCK_M = 1024

_NC = 2
_NS = 16
_L = 16
_NW = _NC * _NS
_ROWS_PER_W = _N_TOKENS // _NW
_GROUPS = _ROWS_PER_W // _L


def _router_body(x_ref, wt_ref, probs_ref, probs_t_ref):
    x = x_ref[...]
    wt = wt_ref[...]
    logits = jax.lax.dot_general(
        x, wt, (((1,), (0,)), ((), ())), preferred_element_type=jnp.float32
    )
    m = jnp.max(logits, axis=-1, keepdims=True)
    e = jnp.exp(logits - m)
    probs = e / jnp.sum(e, axis=-1, keepdims=True)
    probs_ref[...] = probs
    nw_blk = _BLOCK_M // _ROWS_PER_W
    probs_t_ref[...] = probs.reshape(
        nw_blk, _ROWS_PER_W, _NUM_EXPERTS
    ).transpose(0, 2, 1)


def _tc_softmax_router(x, wt):
    grid = (_N_TOKENS // _BLOCK_M,)
    return pl.pallas_call(
        _router_body,
        grid=grid,
        in_specs=[
            pl.BlockSpec((_BLOCK_M, _HIDDEN), lambda i: (i, 0)),
            pl.BlockSpec((_HIDDEN, _NUM_EXPERTS), lambda i: (0, 0)),
        ],
        out_specs=[
            pl.BlockSpec((_BLOCK_M, _NUM_EXPERTS), lambda i: (i, 0)),
            pl.BlockSpec(
                (_BLOCK_M // _ROWS_PER_W, _NUM_EXPERTS, _ROWS_PER_W),
                lambda i: (i, 0, 0),
            ),
        ],
        out_shape=[
            jax.ShapeDtypeStruct((_N_TOKENS, _NUM_EXPERTS), jnp.float32),
            jax.ShapeDtypeStruct(
                (_NW, _NUM_EXPERTS, _ROWS_PER_W), jnp.float32
            ),
        ],
    )(x, wt)


_sc_mesh = plsc.VectorSubcoreMesh(
    core_axis_name="c", subcore_axis_name="s", num_cores=_NC, num_subcores=_NS
)


@functools.partial(
    pl.kernel,
    out_type=[
        jax.ShapeDtypeStruct((_TOP_K, _N_TOKENS), jnp.float32),
        jax.ShapeDtypeStruct((_TOP_K, _N_TOKENS), jnp.int32),
    ],
    mesh=_sc_mesh,
    scratch_types=[
        pltpu.VMEM((_NUM_EXPERTS, _ROWS_PER_W), jnp.float32),
        pltpu.VMEM((_TOP_K, _ROWS_PER_W), jnp.float32),
        pltpu.VMEM((_TOP_K, _ROWS_PER_W), jnp.int32),
    ],
)
def _sc_topk(probs_t_hbm, out_v_hbm, out_i_hbm, slab, val_buf, idx_buf):
    wid = lax.axis_index("s") * _NC + lax.axis_index("c")
    base = wid * _ROWS_PER_W
    pltpu.sync_copy(probs_t_hbm.at[wid], slab)

    @plsc.parallel_loop(0, _GROUPS, unroll=1)
    def group_body(g):
        col = g * _L

        mxs = []
        total = jnp.zeros((_L,), jnp.float32)
        mx = None
        for r in range(_TOP_K):
            vs = [slab[e, pl.ds(col, _L)] for e in range(_NUM_EXPERTS)]
            if r > 0:
                vs = [jnp.where(v < mx, v, -1.0) for v in vs]
            ids = list(range(_NUM_EXPERTS))
            lvl_v = [
                jnp.maximum(vs[i], vs[i + 1]) for i in range(0, _NUM_EXPERTS, 2)
            ]
            lvl_i = [
                jnp.where(vs[i] >= vs[i + 1], ids[i], ids[i + 1])
                for i in range(0, _NUM_EXPERTS, 2)
            ]
            while len(lvl_v) > 1:
                nv, ni = [], []
                for i in range(0, len(lvl_v), 2):
                    ge = lvl_v[i] >= lvl_v[i + 1]
                    nv.append(jnp.maximum(lvl_v[i], lvl_v[i + 1]))
                    ni.append(jnp.where(ge, lvl_i[i], lvl_i[i + 1]))
                lvl_v, lvl_i = nv, ni
            mx, ix = lvl_v[0], lvl_i[0]
            idx_buf[r, pl.ds(col, _L)] = ix
            mxs.append(mx)
            total = total + mx

        inv = 1.0 / total
        for r in range(_TOP_K):
            val_buf[r, pl.ds(col, _L)] = mxs[r] * inv

    pltpu.sync_copy(val_buf, out_v_hbm.at[:, pl.ds(base, _ROWS_PER_W)])
    pltpu.sync_copy(idx_buf, out_i_hbm.at[:, pl.ds(base, _ROWS_PER_W)])


@jax.jit
def kernel(hidden_states, weight):
    x = hidden_states.reshape(-1, _HIDDEN)
    wt = weight.T
    probs, probs_t = _tc_softmax_router(x, wt)
    vals_t, idx_t = _sc_topk(probs_t)
    return (probs, vals_t.T, idx_t.T)
